# baseline (speedup 1.0000x reference)
import functools

import jax
import jax.numpy as jnp
from jax import lax
from jax.experimental import pallas as pl
from jax.experimental.pallas import tpu as pltpu
from jax.experimental.pallas import tpu_sc as plsc

NN = 2048
BS = 4
NODES = NN * BS
HEADS = 4
GD = 16
HID = HEADS * GD
EDGES = 262144
NWORK = 32
EPW = EDGES // NWORK
CHUNK = 128
NCHUNK = EPW // CHUNK
SW = 128
DW = 128


def _fc_body(xe_ref, w_ref, b_ref, o_ref):
    o_ref[...] = jnp.dot(xe_ref[...], w_ref[...].T,
                         preferred_element_type=jnp.float32) + b_ref[...]


def _fc(xe, fc_w, fc_b):
    return pl.pallas_call(
        _fc_body,
        out_shape=jax.ShapeDtypeStruct((NODES, 3 * 4), jnp.float32),
    )(xe, fc_w, fc_b)


def _prep_body(h_ref, w_ref, asb_ref, adb_ref, rep_ref,
               s_ref, d_ref, selfm_ref):
    hW = jnp.dot(h_ref[...], w_ref[...].T, preferred_element_type=jnp.float32)
    a1 = jnp.dot(hW, asb_ref[...].T, preferred_element_type=jnp.float32)
    a2 = jnp.dot(hW, adb_ref[...].T, preferred_element_type=jnp.float32)
    rep = rep_ref[...]
    u1 = jnp.exp(a1)
    u2 = jnp.exp(0.2 * a1)
    v1 = jnp.exp(a2)
    v2 = jnp.exp(0.2 * a2)
    s_ref[...] = jnp.concatenate(
        [hW, u1, u2, jnp.zeros((NODES, SW - HID - 8), jnp.float32)], axis=1)
    d_ref[...] = v2
    u1b = jnp.dot(u1, rep, preferred_element_type=jnp.float32)
    u2b = jnp.dot(u2, rep, preferred_element_type=jnp.float32)
    v1b = jnp.dot(v1, rep, preferred_element_type=jnp.float32)
    v2b = jnp.dot(v2, rep, preferred_element_type=jnp.float32)
    selfm_ref[...] = jnp.maximum(u1b * v1b, u2b * v2b)


def _prep(h, W, asb, adb, rep):
    return pl.pallas_call(
        _prep_body,
        out_shape=(
            jax.ShapeDtypeStruct((NODES, SW), jnp.float32),
            jax.ShapeDtypeStruct((NODES, 4), jnp.float32),
            jax.ShapeDtypeStruct((NODES, HID), jnp.float32),
        ),
    )(h, W, asb, adb, rep)


def _dyngather(x, idx):
    return lax.gather(
        x, idx[:, None],
        lax.GatherDimensionNumbers(offset_dims=(), collapsed_slice_dims=(0,),
                                   start_index_map=(0,)),
        (1,), mode=lax.GatherScatterMode.PROMISE_IN_BOUNDS)


def _sc_edge_body(s_hbm, d_hbm, src_hbm, dst_hbm, out_hbm,
                  sidx, didx, vtab, sbuf, acc, sem_s):
    cid = lax.axis_index("c")
    sid = lax.axis_index("s")
    wid = cid * 16 + sid

    iota = lax.iota(jnp.int32, 16)
    shift4 = jnp.minimum(iota + 4, 15)
    pat4 = lax.rem(iota, 4)

    zero16 = jnp.zeros((16,), jnp.float32)

    def _zrow(e, carry):
        for j in range(8):
            sbuf[e, pl.ds(16 * j, 16)] = zero16
        return carry

    lax.fori_loop(0, CHUNK, _zrow, 0)
    for t in range(512 // CHUNK):
        pltpu.sync_copy(sbuf, acc.at[pl.ds(sid * 512 + t * CHUNK, CHUNK)])
    plsc.subcore_barrier()

    pltpu.sync_copy(src_hbm.at[wid], sidx)
    pltpu.sync_copy(dst_hbm.at[wid], didx)
    pltpu.sync_copy(d_hbm, vtab)

    def _chunk(c, carry):
        pltpu.async_copy(s_hbm.at[sidx.at[c]], sbuf, sem_s).wait()
        dst_c = didx.at[c]

        def _grp(g, gcarry):
            ebase = 16 * g
            dst16 = didx[c, pl.ds(ebase, 16)]
            for l in range(16):
                e = ebase + l
                d = dst16[l]
                start = jnp.minimum(d * 4, 4 * NODES - 16)
                sh = d * 4 - start
                dvl = vtab[pl.ds(start, 16)]
                t = _dyngather(dvl, pat4 + sh)
                t2 = t * t
                dv = jnp.where(iota < 4, t2 * t2 * t, t)
                su = sbuf[e, pl.ds(HID, 16)]
                prod = su * dv
                m = jnp.maximum(prod, _dyngather(prod, shift4))
                sbuf[e, pl.ds(HID, 16)] = m
                for j in range(4):
                    exj = _dyngather(m, jnp.full((16,), j, jnp.int32))
                    sbuf[e, pl.ds(16 * j, 16)] = (
                        sbuf[e, pl.ds(16 * j, 16)] * exj)
            return gcarry

        lax.fori_loop(0, CHUNK // 16, _grp, 0)
        pltpu.sync_copy(sbuf, acc.at[dst_c], add=True)
        return carry

    lax.fori_loop(0, NCHUNK, _chunk, 0)
    plsc.subcore_barrier()

    pltpu.sync_copy(acc.at[pl.ds(sid * 512, 512)],
                    out_hbm.at[cid, pl.ds(sid * 512, 512)])


def _sc_edge(s_tab, d_tab, src_w, dst_w):
    mesh = plsc.VectorSubcoreMesh(core_axis_name="c", subcore_axis_name="s")
    f = pl.kernel(
        _sc_edge_body,
        out_type=jax.ShapeDtypeStruct((2, NODES, DW), jnp.float32),
        mesh=mesh,
        scratch_types=[
            pltpu.VMEM((NCHUNK, CHUNK), jnp.int32),
            pltpu.VMEM((NCHUNK, CHUNK), jnp.int32),
            pltpu.VMEM((4 * NODES,), jnp.float32),
            pltpu.VMEM((CHUNK, SW), jnp.float32),
            pltpu.VMEM_SHARED((NODES, DW), jnp.float32),
            pltpu.SemaphoreType.DMA,
        ],
    )
    return f(s_tab, d_tab, src_w, dst_w)


def _combine_body(p_ref, s_ref, selfm_ref, rep_ref, b_ref, o_ref):
    num = p_ref[0, :, :HID] + p_ref[1, :, :HID] + s_ref[:, :HID] * selfm_ref[...]
    den4 = p_ref[0, :, HID:HID + 4] + p_ref[1, :, HID:HID + 4]
    den = jnp.dot(den4, rep_ref[...],
                  preferred_element_type=jnp.float32) + selfm_ref[...]
    o = num / (den + 1e-16) + b_ref[...]
    o_ref[...] = jnp.where(o > 0, o, 0.01 * o)


def _combine(p, s_tab, selfm, rep, b):
    return pl.pallas_call(
        _combine_body,
        out_shape=jax.ShapeDtypeStruct((NODES, HID), jnp.float32),
    )(p, s_tab, selfm, rep, b)


def _mha_body(x_ref, wi_ref, bi_ref, o_ref):
    x = x_ref[0]
    qkv = jnp.dot(x, wi_ref[...].T, preferred_element_type=jnp.float32) + bi_ref[...]
    for h in range(HEADS):
        q = qkv[:, 16 * h:16 * h + 16]
        k = qkv[:, HID + 16 * h:HID + 16 * h + 16]
        v = qkv[:, 2 * HID + 16 * h:2 * HID + 16 * h + 16]
        sc = jnp.dot(q, k.T, preferred_element_type=jnp.float32) * 0.25
        m = jnp.max(sc, axis=-1, keepdims=True)
        e = jnp.exp(sc - m)
        at = e / jnp.sum(e, axis=-1, keepdims=True)
        o_ref[0, :, 16 * h:16 * h + 16] = jnp.dot(
            at, v, preferred_element_type=jnp.float32)


def _mha(x4, in_w, in_b):
    return pl.pallas_call(
        _mha_body,
        grid=(BS,),
        in_specs=[
            pl.BlockSpec((1, NN, HID), lambda b: (b, 0, 0)),
            pl.BlockSpec((3 * HID, HID), lambda b: (0, 0)),
            pl.BlockSpec((3 * HID,), lambda b: (0,)),
        ],
        out_specs=pl.BlockSpec((1, NN, HID), lambda b: (b, 0, 0)),
        out_shape=jax.ShapeDtypeStruct((BS, NN, HID), jnp.float32),
    )(x4, in_w, in_b)


def _proj_body(o_ref, dow_ref, dob_ref, pw_ref, pb_ref, p_ref):
    y = jnp.dot(o_ref[...], dow_ref[...].T,
                preferred_element_type=jnp.float32) + dob_ref[...]
    p_ref[...] = jnp.dot(y, pw_ref[...].T,
                         preferred_element_type=jnp.float32) + pb_ref[...]


def _outmm_body(pz_ref, ow_ref, ob_ref, z_ref):
    z_ref[...] = jnp.dot(pz_ref[...], ow_ref[...].T,
                         preferred_element_type=jnp.float32) + ob_ref[...]


def _final(o, dow, dob, pw, pb, ow, ob):
    p = pl.pallas_call(
        _proj_body,
        out_shape=jax.ShapeDtypeStruct((NODES, 4), jnp.float32),
    )(o, dow, dob, pw, pb)
    return pl.pallas_call(
        _outmm_body,
        out_shape=jax.ShapeDtypeStruct((BS, 128), jnp.float32),
    )(p.reshape(BS, NN * 4), ow, ob)


def kernel(x, edge_index, batch, emb, fc_w, fc_b,
           g0_W, g0_as, g0_ad, g0_b,
           g1_W, g1_as, g1_ad, g1_b,
           g2_W, g2_as, g2_ad, g2_b,
           g3_W, g3_as, g3_ad, g3_b,
           dec_in_w, dec_in_b, dec_out_w, dec_out_b,
           proj_w, proj_b, out_w, out_b):
    embt = jnp.broadcast_to(emb[None], (BS, NN, 4)).reshape(NODES, 4)
    xe = jnp.concatenate([x, embt], axis=1)
    src_w = edge_index[0].reshape(NWORK, NCHUNK, CHUNK)
    dst_w = edge_index[1].reshape(NWORK, NCHUNK, CHUNK)
    eye = jnp.eye(HEADS, dtype=jnp.float32)
    rep = (eye[:, :, None] *
           jnp.ones((1, HEADS, GD), jnp.float32)).reshape(HEADS, HID)

    h = _fc(xe, fc_w, fc_b)
    for W, a_s, a_d, b in ((g0_W, g0_as, g0_ad, g0_b),
                           (g1_W, g1_as, g1_ad, g1_b),
                           (g2_W, g2_as, g2_ad, g2_b),
                           (g3_W, g3_as, g3_ad, g3_b)):
        asb = (eye[:, :, None] * a_s[None, :, :]).reshape(HEADS, HID)
        adb = (eye[:, :, None] * a_d[None, :, :]).reshape(HEADS, HID)
        s_tab, d_tab, selfm = _prep(h, W, asb, adb, rep)
        p = _sc_edge(s_tab, d_tab.reshape(4 * NODES), src_w, dst_w)
        h = _combine(p, s_tab, selfm, rep, b)

    o4 = _mha(h.reshape(BS, NN, HID), dec_in_w, dec_in_b)
    return _final(o4.reshape(NODES, HID), dec_out_w, dec_out_b,
                  proj_w, proj_b, out_w, out_b)


# speedup vs baseline: 1.1146x; 1.1146x over previous
import functools

import jax
import jax.numpy as jnp
from jax import lax
from jax.experimental import pallas as pl
from jax.experimental.pallas import tpu as pltpu
from jax.experimental.pallas import tpu_sc as plsc

NN = 2048
BS = 4
NODES = NN * BS
HEADS = 4
GD = 16
HID = HEADS * GD
EDGES = 262144
NWORK = 32
EPW = EDGES // NWORK
CHUNK = 64
NCHUNK = EPW // CHUNK
SW = 128
DW = 128


def _prep_tables(h, w_ref, asb_ref, adb_ref, rep, s_ref, d_ref, selfm_ref):
    hW = jnp.dot(h, w_ref[...].T, preferred_element_type=jnp.float32)
    a1 = jnp.dot(hW, asb_ref[...].T, preferred_element_type=jnp.float32)
    a2 = jnp.dot(hW, adb_ref[...].T, preferred_element_type=jnp.float32)
    u1 = jnp.exp(a1)
    u2 = jnp.exp(0.2 * a1)
    v1 = jnp.exp(a2)
    v2 = jnp.exp(0.2 * a2)
    s_ref[...] = jnp.concatenate(
        [hW, u1, u2, jnp.zeros((NODES, SW - HID - 8), jnp.float32)], axis=1)
    d_ref[...] = jnp.concatenate(
        [v1, v2, jnp.zeros((NODES, DW - 8), jnp.float32)], axis=1)
    u1b = jnp.dot(u1, rep, preferred_element_type=jnp.float32)
    u2b = jnp.dot(u2, rep, preferred_element_type=jnp.float32)
    v1b = jnp.dot(v1, rep, preferred_element_type=jnp.float32)
    v2b = jnp.dot(v2, rep, preferred_element_type=jnp.float32)
    selfm_ref[...] = jnp.maximum(u1b * v1b, u2b * v2b)


_PREP_OUT = (
    jax.ShapeDtypeStruct((NODES, SW), jnp.float32),
    jax.ShapeDtypeStruct((NODES, DW), jnp.float32),
    jax.ShapeDtypeStruct((NODES, HID), jnp.float32),
)


def _prep0_body(xe_ref, fcw_ref, fcb_ref, w_ref, asb_ref, adb_ref, rep_ref,
                s_ref, d_ref, selfm_ref):
    h = jnp.dot(xe_ref[...], fcw_ref[...].T,
                preferred_element_type=jnp.float32) + fcb_ref[...]
    _prep_tables(h, w_ref, asb_ref, adb_ref, rep_ref[...],
                 s_ref, d_ref, selfm_ref)


def _prep0(xe, fc_w, fc_b, W, asb, adb, rep):
    return pl.pallas_call(
        _prep0_body, out_shape=_PREP_OUT,
    )(xe, fc_w, fc_b, W, asb, adb, rep)


def _combine_h(p_ref, s_ref, selfm_ref, rep, b_ref):
    num = (p_ref[0, :, :HID] + p_ref[1, :, :HID]
           + s_ref[:, :HID] * selfm_ref[...])
    den4 = p_ref[0, :, HID:HID + 4] + p_ref[1, :, HID:HID + 4]
    den = jnp.dot(den4, rep, preferred_element_type=jnp.float32) + selfm_ref[...]
    o = num / (den + 1e-16) + b_ref[...]
    return jnp.where(o > 0, o, 0.01 * o)


def _prepn_body(p_ref, sp_ref, selfmp_ref, b_ref, w_ref, asb_ref, adb_ref,
                rep_ref, s_ref, d_ref, selfm_ref):
    rep = rep_ref[...]
    h = _combine_h(p_ref, sp_ref, selfmp_ref, rep, b_ref)
    _prep_tables(h, w_ref, asb_ref, adb_ref, rep, s_ref, d_ref, selfm_ref)


def _prepn(p, s_prev, selfm_prev, b, W, asb, adb, rep):
    return pl.pallas_call(
        _prepn_body, out_shape=_PREP_OUT,
    )(p, s_prev, selfm_prev, b, W, asb, adb, rep)


def _dyngather(x, idx):
    return lax.gather(
        x, idx[:, None],
        lax.GatherDimensionNumbers(offset_dims=(), collapsed_slice_dims=(0,),
                                   start_index_map=(0,)),
        (1,), mode=lax.GatherScatterMode.PROMISE_IN_BOUNDS)


def _sc_edge_body(s_hbm, d_hbm, src_hbm, dst_hbm, out_hbm,
                  sidx, didx, sbuf, dbuf, obuf, acc, sem_s, sem_d):
    cid = lax.axis_index("c")
    sid = lax.axis_index("s")
    wid = cid * 16 + sid

    iota = lax.iota(jnp.int32, 16)
    shift4 = jnp.minimum(iota + 4, 15)

    zero16 = jnp.zeros((16,), jnp.float32)

    def _zrow(e, carry):
        for j in range(8):
            obuf[e, pl.ds(16 * j, 16)] = zero16
        return carry

    lax.fori_loop(0, CHUNK, _zrow, 0)
    for t in range(512 // CHUNK):
        pltpu.sync_copy(obuf, acc.at[pl.ds(sid * 512 + t * CHUNK, CHUNK)])
    plsc.subcore_barrier()

    pltpu.sync_copy(src_hbm.at[wid], sidx)
    pltpu.sync_copy(dst_hbm.at[wid], didx)

    def _chunk(c, carry):
        cp_s = pltpu.async_copy(s_hbm.at[sidx.at[c]], sbuf, sem_s)
        cp_d = pltpu.async_copy(d_hbm.at[didx.at[c]], dbuf, sem_d)
        cp_s.wait()
        cp_d.wait()

        def _grp(g, gcarry):
            ebase = 16 * g
            for l in range(16):
                e = ebase + l
                dv = dbuf[e, pl.ds(0, 16)]
                su = sbuf[e, pl.ds(HID, 16)]
                prod = su * dv
                m = jnp.maximum(prod, _dyngather(prod, shift4))
                obuf[e, pl.ds(HID, 16)] = m
                for j in range(4):
                    exj = _dyngather(m, jnp.full((16,), j, jnp.int32))
                    obuf[e, pl.ds(16 * j, 16)] = (
                        sbuf[e, pl.ds(16 * j, 16)] * exj)
            return gcarry

        lax.fori_loop(0, CHUNK // 16, _grp, 0)
        pltpu.sync_copy(obuf, acc.at[didx.at[c]], add=True)
        return carry

    lax.fori_loop(0, NCHUNK, _chunk, 0)
    plsc.subcore_barrier()

    pltpu.sync_copy(acc.at[pl.ds(sid * 512, 512)],
                    out_hbm.at[cid, pl.ds(sid * 512, 512)])


def _sc_edge(s_tab, d_tab, src_w, dst_w):
    mesh = plsc.VectorSubcoreMesh(core_axis_name="c", subcore_axis_name="s")
    f = pl.kernel(
        _sc_edge_body,
        out_type=jax.ShapeDtypeStruct((2, NODES, DW), jnp.float32),
        mesh=mesh,
        scratch_types=[
            pltpu.VMEM((NCHUNK, CHUNK), jnp.int32),
            pltpu.VMEM((NCHUNK, CHUNK), jnp.int32),
            pltpu.VMEM((CHUNK, SW), jnp.float32),
            pltpu.VMEM((CHUNK, DW), jnp.float32),
            pltpu.VMEM((CHUNK, DW), jnp.float32),
            pltpu.VMEM_SHARED((NODES, DW), jnp.float32),
            pltpu.SemaphoreType.DMA,
            pltpu.SemaphoreType.DMA,
        ],
    )
    return f(s_tab, d_tab, src_w, dst_w)


def _combine_body(p_ref, s_ref, selfm_ref, rep_ref, b_ref, o_ref):
    o_ref[...] = _combine_h(p_ref, s_ref, selfm_ref, rep_ref[...], b_ref)


def _combine(p, s_tab, selfm, rep, b):
    return pl.pallas_call(
        _combine_body,
        out_shape=jax.ShapeDtypeStruct((NODES, HID), jnp.float32),
    )(p, s_tab, selfm, rep, b)


def _mha_body(x_ref, wi_ref, bi_ref, o_ref):
    x = x_ref[0]
    qkv = jnp.dot(x, wi_ref[...].T, preferred_element_type=jnp.float32) + bi_ref[...]
    for h in range(HEADS):
        q = qkv[:, 16 * h:16 * h + 16]
        k = qkv[:, HID + 16 * h:HID + 16 * h + 16]
        v = qkv[:, 2 * HID + 16 * h:2 * HID + 16 * h + 16]
        sc = jnp.dot(q, k.T, preferred_element_type=jnp.float32) * 0.25
        m = jnp.max(sc, axis=-1, keepdims=True)
        e = jnp.exp(sc - m)
        at = e / jnp.sum(e, axis=-1, keepdims=True)
        o_ref[0, :, 16 * h:16 * h + 16] = jnp.dot(
            at, v, preferred_element_type=jnp.float32)


def _mha(x4, in_w, in_b):
    return pl.pallas_call(
        _mha_body,
        grid=(BS,),
        in_specs=[
            pl.BlockSpec((1, NN, HID), lambda b: (b, 0, 0)),
            pl.BlockSpec((3 * HID, HID), lambda b: (0, 0)),
            pl.BlockSpec((3 * HID,), lambda b: (0,)),
        ],
        out_specs=pl.BlockSpec((1, NN, HID), lambda b: (b, 0, 0)),
        out_shape=jax.ShapeDtypeStruct((BS, NN, HID), jnp.float32),
    )(x4, in_w, in_b)


def _proj_body(o_ref, dow_ref, dob_ref, pw_ref, pb_ref, p_ref):
    y = jnp.dot(o_ref[...], dow_ref[...].T,
                preferred_element_type=jnp.float32) + dob_ref[...]
    p_ref[...] = jnp.dot(y, pw_ref[...].T,
                         preferred_element_type=jnp.float32) + pb_ref[...]


def _outmm_body(pz_ref, ow_ref, ob_ref, z_ref):
    z_ref[...] = jnp.dot(pz_ref[...], ow_ref[...].T,
                         preferred_element_type=jnp.float32) + ob_ref[...]


def _final(o, dow, dob, pw, pb, ow, ob):
    p = pl.pallas_call(
        _proj_body,
        out_shape=jax.ShapeDtypeStruct((NODES, 4), jnp.float32),
    )(o, dow, dob, pw, pb)
    return pl.pallas_call(
        _outmm_body,
        out_shape=jax.ShapeDtypeStruct((BS, 128), jnp.float32),
    )(p.reshape(BS, NN * 4), ow, ob)


def kernel(x, edge_index, batch, emb, fc_w, fc_b,
           g0_W, g0_as, g0_ad, g0_b,
           g1_W, g1_as, g1_ad, g1_b,
           g2_W, g2_as, g2_ad, g2_b,
           g3_W, g3_as, g3_ad, g3_b,
           dec_in_w, dec_in_b, dec_out_w, dec_out_b,
           proj_w, proj_b, out_w, out_b):
    embt = jnp.broadcast_to(emb[None], (BS, NN, 4)).reshape(NODES, 4)
    xe = jnp.concatenate([x, embt], axis=1)
    src_w = edge_index[0].reshape(NWORK, NCHUNK, CHUNK)
    dst_w = edge_index[1].reshape(NWORK, NCHUNK, CHUNK)
    eye = jnp.eye(HEADS, dtype=jnp.float32)
    rep = (eye[:, :, None] *
           jnp.ones((1, HEADS, GD), jnp.float32)).reshape(HEADS, HID)

    layers = ((g0_W, g0_as, g0_ad, g0_b), (g1_W, g1_as, g1_ad, g1_b),
              (g2_W, g2_as, g2_ad, g2_b), (g3_W, g3_as, g3_ad, g3_b))
    asbs = [(eye[:, :, None] * a_s[None, :, :]).reshape(HEADS, HID)
            for (_, a_s, _, _) in layers]
    adbs = [(eye[:, :, None] * a_d[None, :, :]).reshape(HEADS, HID)
            for (_, _, a_d, _) in layers]

    s_tab, d_tab, selfm = _prep0(xe, fc_w, fc_b, layers[0][0],
                                 asbs[0], adbs[0], rep)
    for i in range(4):
        p = _sc_edge(s_tab, d_tab, src_w, dst_w)
        b = layers[i][3]
        if i < 3:
            s_tab, d_tab, selfm = _prepn(p, s_tab, selfm, b,
                                         layers[i + 1][0], asbs[i + 1],
                                         adbs[i + 1], rep)
        else:
            h = _combine(p, s_tab, selfm, rep, b)

    o4 = _mha(h.reshape(BS, NN, HID), dec_in_w, dec_in_b)
    return _final(o4.reshape(NODES, HID), dec_out_w, dec_out_b,
                  proj_w, proj_b, out_w, out_b)


# speedup vs baseline: 1.6362x; 1.4679x over previous
import functools

import jax
import jax.numpy as jnp
from jax import lax
from jax.experimental import pallas as pl
from jax.experimental.pallas import tpu as pltpu
from jax.experimental.pallas import tpu_sc as plsc

NN = 2048
BS = 4
NODES = NN * BS
HEADS = 4
GD = 16
HID = HEADS * GD
EDGES = 262144
NWORK = 32
EPW = EDGES // NWORK
CHUNK = 64
NCHUNK = EPW // CHUNK
SW = 128
DW = 128


def _prep_tables(h, w_ref, asb_ref, adb_ref, rep, s_ref, d_ref, selfm_ref):
    hW = jnp.dot(h, w_ref[...].T, preferred_element_type=jnp.float32)
    a1 = jnp.dot(hW, asb_ref[...].T, preferred_element_type=jnp.float32)
    a2 = jnp.dot(hW, adb_ref[...].T, preferred_element_type=jnp.float32)
    u1 = jnp.exp(a1)
    u2 = jnp.exp(0.2 * a1)
    v1 = jnp.exp(a2)
    v2 = jnp.exp(0.2 * a2)
    s_ref[...] = jnp.concatenate(
        [hW, u1, u2, jnp.zeros((NODES, SW - HID - 8), jnp.float32)], axis=1)
    d_ref[...] = jnp.concatenate(
        [v1, v2, jnp.zeros((NODES, DW - 8), jnp.float32)], axis=1)
    u1b = jnp.dot(u1, rep, preferred_element_type=jnp.float32)
    u2b = jnp.dot(u2, rep, preferred_element_type=jnp.float32)
    v1b = jnp.dot(v1, rep, preferred_element_type=jnp.float32)
    v2b = jnp.dot(v2, rep, preferred_element_type=jnp.float32)
    selfm_ref[...] = jnp.maximum(u1b * v1b, u2b * v2b)


_PREP_OUT = (
    jax.ShapeDtypeStruct((NODES, SW), jnp.float32),
    jax.ShapeDtypeStruct((NODES, DW), jnp.float32),
    jax.ShapeDtypeStruct((NODES, HID), jnp.float32),
)


def _prep0_body(xe_ref, fcw_ref, fcb_ref, w_ref, asb_ref, adb_ref, rep_ref,
                s_ref, d_ref, selfm_ref):
    h = jnp.dot(xe_ref[...], fcw_ref[...].T,
                preferred_element_type=jnp.float32) + fcb_ref[...]
    _prep_tables(h, w_ref, asb_ref, adb_ref, rep_ref[...],
                 s_ref, d_ref, selfm_ref)


def _prep0(xe, fc_w, fc_b, W, asb, adb, rep):
    return pl.pallas_call(
        _prep0_body, out_shape=_PREP_OUT,
    )(xe, fc_w, fc_b, W, asb, adb, rep)


def _combine_h(p_ref, s_ref, selfm_ref, rep, b_ref):
    num = (p_ref[0, :, :HID] + p_ref[1, :, :HID]
           + s_ref[:, :HID] * selfm_ref[...])
    den4 = p_ref[0, :, HID:HID + 4] + p_ref[1, :, HID:HID + 4]
    den = jnp.dot(den4, rep, preferred_element_type=jnp.float32) + selfm_ref[...]
    o = num / (den + 1e-16) + b_ref[...]
    return jnp.where(o > 0, o, 0.01 * o)


def _prepn_body(p_ref, sp_ref, selfmp_ref, b_ref, w_ref, asb_ref, adb_ref,
                rep_ref, s_ref, d_ref, selfm_ref):
    rep = rep_ref[...]
    h = _combine_h(p_ref, sp_ref, selfmp_ref, rep, b_ref)
    _prep_tables(h, w_ref, asb_ref, adb_ref, rep, s_ref, d_ref, selfm_ref)


def _prepn(p, s_prev, selfm_prev, b, W, asb, adb, rep):
    return pl.pallas_call(
        _prepn_body, out_shape=_PREP_OUT,
    )(p, s_prev, selfm_prev, b, W, asb, adb, rep)


def _dyngather(x, idx):
    return lax.gather(
        x, idx[:, None],
        lax.GatherDimensionNumbers(offset_dims=(), collapsed_slice_dims=(0,),
                                   start_index_map=(0,)),
        (1,), mode=lax.GatherScatterMode.PROMISE_IN_BOUNDS)


def _sc_edge_body(s_hbm, d_hbm, e_hbm, out_hbm,
                  etile, schunk, dchunk, sbuf0, sbuf1, dbuf0, dbuf1,
                  obuf0, obuf1, acc,
                  sem_s0, sem_s1, sem_d0, sem_d1, sem_o0, sem_o1):
    cid = lax.axis_index("c")
    sid = lax.axis_index("s")
    wid = cid * 16 + sid
    sbufs = (sbuf0, sbuf1)
    dbufs = (dbuf0, dbuf1)
    obufs = (obuf0, obuf1)
    sem_ss = (sem_s0, sem_s1)
    sem_ds = (sem_d0, sem_d1)
    sem_os = (sem_o0, sem_o1)

    iota = lax.iota(jnp.int32, 16)
    shift4 = jnp.minimum(iota + 4, 15)

    zero16 = jnp.zeros((16,), jnp.float32)

    def _zrow(e, carry):
        for j in range(8):
            obuf0[e, pl.ds(16 * j, 16)] = zero16
        return carry

    lax.fori_loop(0, CHUNK, _zrow, 0)
    for t in range(512 // CHUNK):
        pltpu.sync_copy(obuf0, acc.at[pl.ds(sid * 512 + t * CHUNK, CHUNK)])
    plsc.subcore_barrier()

    pltpu.sync_copy(e_hbm.at[wid], etile)

    def _unpack(q, row, off):
        for k in range(4):
            ev = etile[row, pl.ds(off + 16 * k, 16)]
            schunk[lax.rem(q, 2), pl.ds(16 * k, 16)] = (
                lax.bitwise_and(ev, 8191))
            dchunk[lax.rem(q, 4), pl.ds(16 * k, 16)] = (
                lax.shift_right_logical(ev, 13))

    def _issue(q, b):
        pltpu.async_copy(s_hbm.at[schunk.at[lax.rem(q, 2)]],
                         sbufs[b], sem_ss[b])
        pltpu.async_copy(d_hbm.at[dchunk.at[lax.rem(q, 4)]],
                         dbufs[b], sem_ds[b])

    _unpack(jnp.int32(0), jnp.int32(0), 0)
    _issue(jnp.int32(0), 0)

    def _outer(c2, carry):
        for b in range(2):
            c = 2 * c2 + b
            nb = 1 - b

            @pl.when(c + 1 < NCHUNK)
            def _():
                nrow = c2 + b
                _unpack(c + 1, nrow, 64 * nb)
                _issue(c + 1, nb)

            @pl.when(c >= 2)
            def _():
                pltpu.make_async_copy(
                    obufs[b], acc.at[dchunk.at[lax.rem(c, 4)]],
                    sem_os[b]).wait()

            pltpu.make_async_copy(
                s_hbm.at[schunk.at[lax.rem(c, 2)]], sbufs[b],
                sem_ss[b]).wait()
            pltpu.make_async_copy(
                d_hbm.at[dchunk.at[lax.rem(c, 4)]], dbufs[b],
                sem_ds[b]).wait()
            sbuf = sbufs[b]
            dbuf = dbufs[b]
            obuf = obufs[b]

            def _grp(g, gcarry):
                ebase = 16 * g
                for l in range(16):
                    e = ebase + l
                    dv = dbuf[e, pl.ds(0, 16)]
                    su = sbuf[e, pl.ds(HID, 16)]
                    prod = su * dv
                    m = jnp.maximum(prod, _dyngather(prod, shift4))
                    obuf[e, pl.ds(HID, 16)] = m
                    for j in range(4):
                        exj = _dyngather(m, jnp.full((16,), j, jnp.int32))
                        obuf[e, pl.ds(16 * j, 16)] = (
                            sbuf[e, pl.ds(16 * j, 16)] * exj)
                return gcarry

            lax.fori_loop(0, CHUNK // 16, _grp, 0)
            pltpu.async_copy(obuf, acc.at[dchunk.at[lax.rem(c, 4)]],
                             sem_os[b], add=True)
        return carry

    lax.fori_loop(0, NCHUNK // 2, _outer, 0)
    for b in range(2):
        c = NCHUNK - 2 + b
        pltpu.make_async_copy(
            obufs[b], acc.at[dchunk.at[lax.rem(jnp.int32(c), 4)]],
            sem_os[b]).wait()
    plsc.subcore_barrier()

    pltpu.sync_copy(acc.at[pl.ds(sid * 512, 512)],
                    out_hbm.at[cid, pl.ds(sid * 512, 512)])


def _sc_edge(s_tab, d_tab, edges_w):
    mesh = plsc.VectorSubcoreMesh(core_axis_name="c", subcore_axis_name="s")
    f = pl.kernel(
        _sc_edge_body,
        out_type=jax.ShapeDtypeStruct((2, NODES, DW), jnp.float32),
        mesh=mesh,
        scratch_types=[
            pltpu.VMEM((NCHUNK // 2, 2 * CHUNK), jnp.int32),
            pltpu.VMEM((2, CHUNK), jnp.int32),
            pltpu.VMEM((4, CHUNK), jnp.int32),
            pltpu.VMEM((CHUNK, SW), jnp.float32),
            pltpu.VMEM((CHUNK, SW), jnp.float32),
            pltpu.VMEM((CHUNK, DW), jnp.float32),
            pltpu.VMEM((CHUNK, DW), jnp.float32),
            pltpu.VMEM((CHUNK, DW), jnp.float32),
            pltpu.VMEM((CHUNK, DW), jnp.float32),
            pltpu.VMEM_SHARED((NODES, DW), jnp.float32),
            pltpu.SemaphoreType.DMA,
            pltpu.SemaphoreType.DMA,
            pltpu.SemaphoreType.DMA,
            pltpu.SemaphoreType.DMA,
            pltpu.SemaphoreType.DMA,
            pltpu.SemaphoreType.DMA,
        ],
    )
    return f(s_tab, d_tab, edges_w)


def _combine_body(p_ref, s_ref, selfm_ref, rep_ref, b_ref, o_ref):
    o_ref[...] = _combine_h(p_ref, s_ref, selfm_ref, rep_ref[...], b_ref)


def _combine(p, s_tab, selfm, rep, b):
    return pl.pallas_call(
        _combine_body,
        out_shape=jax.ShapeDtypeStruct((NODES, HID), jnp.float32),
    )(p, s_tab, selfm, rep, b)


def _mha_body(x_ref, wi_ref, bi_ref, o_ref):
    x = x_ref[0]
    qkv = jnp.dot(x, wi_ref[...].T, preferred_element_type=jnp.float32) + bi_ref[...]
    for h in range(HEADS):
        q = qkv[:, 16 * h:16 * h + 16]
        k = qkv[:, HID + 16 * h:HID + 16 * h + 16]
        v = qkv[:, 2 * HID + 16 * h:2 * HID + 16 * h + 16]
        sc = jnp.dot(q, k.T, preferred_element_type=jnp.float32) * 0.25
        m = jnp.max(sc, axis=-1, keepdims=True)
        e = jnp.exp(sc - m)
        at = e / jnp.sum(e, axis=-1, keepdims=True)
        o_ref[0, :, 16 * h:16 * h + 16] = jnp.dot(
            at, v, preferred_element_type=jnp.float32)


def _mha(x4, in_w, in_b):
    return pl.pallas_call(
        _mha_body,
        grid=(BS,),
        in_specs=[
            pl.BlockSpec((1, NN, HID), lambda b: (b, 0, 0)),
            pl.BlockSpec((3 * HID, HID), lambda b: (0, 0)),
            pl.BlockSpec((3 * HID,), lambda b: (0,)),
        ],
        out_specs=pl.BlockSpec((1, NN, HID), lambda b: (b, 0, 0)),
        out_shape=jax.ShapeDtypeStruct((BS, NN, HID), jnp.float32),
    )(x4, in_w, in_b)


def _proj_body(o_ref, dow_ref, dob_ref, pw_ref, pb_ref, p_ref):
    y = jnp.dot(o_ref[...], dow_ref[...].T,
                preferred_element_type=jnp.float32) + dob_ref[...]
    p_ref[...] = jnp.dot(y, pw_ref[...].T,
                         preferred_element_type=jnp.float32) + pb_ref[...]


def _outmm_body(pz_ref, ow_ref, ob_ref, z_ref):
    z_ref[...] = jnp.dot(pz_ref[...], ow_ref[...].T,
                         preferred_element_type=jnp.float32) + ob_ref[...]


def _final(o, dow, dob, pw, pb, ow, ob):
    p = pl.pallas_call(
        _proj_body,
        out_shape=jax.ShapeDtypeStruct((NODES, 4), jnp.float32),
    )(o, dow, dob, pw, pb)
    return pl.pallas_call(
        _outmm_body,
        out_shape=jax.ShapeDtypeStruct((BS, 128), jnp.float32),
    )(p.reshape(BS, NN * 4), ow, ob)


def kernel(x, edge_index, batch, emb, fc_w, fc_b,
           g0_W, g0_as, g0_ad, g0_b,
           g1_W, g1_as, g1_ad, g1_b,
           g2_W, g2_as, g2_ad, g2_b,
           g3_W, g3_as, g3_ad, g3_b,
           dec_in_w, dec_in_b, dec_out_w, dec_out_b,
           proj_w, proj_b, out_w, out_b):
    embt = jnp.broadcast_to(emb[None], (BS, NN, 4)).reshape(NODES, 4)
    xe = jnp.concatenate([x, embt], axis=1)
    edges_w = (edge_index[0] | (edge_index[1] << 13)).reshape(
        NWORK, NCHUNK // 2, 2 * CHUNK)
    eye = jnp.eye(HEADS, dtype=jnp.float32)
    rep = (eye[:, :, None] *
           jnp.ones((1, HEADS, GD), jnp.float32)).reshape(HEADS, HID)

    layers = ((g0_W, g0_as, g0_ad, g0_b), (g1_W, g1_as, g1_ad, g1_b),
              (g2_W, g2_as, g2_ad, g2_b), (g3_W, g3_as, g3_ad, g3_b))
    asbs = [(eye[:, :, None] * a_s[None, :, :]).reshape(HEADS, HID)
            for (_, a_s, _, _) in layers]
    adbs = [(eye[:, :, None] * a_d[None, :, :]).reshape(HEADS, HID)
            for (_, _, a_d, _) in layers]

    s_tab, d_tab, selfm = _prep0(xe, fc_w, fc_b, layers[0][0],
                                 asbs[0], adbs[0], rep)
    for i in range(4):
        p = _sc_edge(s_tab, d_tab, edges_w)
        b = layers[i][3]
        if i < 3:
            s_tab, d_tab, selfm = _prepn(p, s_tab, selfm, b,
                                         layers[i + 1][0], asbs[i + 1],
                                         adbs[i + 1], rep)
        else:
            h = _combine(p, s_tab, selfm, rep, b)

    o4 = _mha(h.reshape(BS, NN, HID), dec_in_w, dec_in_b)
    return _final(o4.reshape(NODES, HID), dec_out_w, dec_out_b,
                  proj_w, proj_b, out_w, out_b)


# speedup vs baseline: 1.6607x; 1.0150x over previous
import functools

import jax
import jax.numpy as jnp
from jax import lax
from jax.experimental import pallas as pl
from jax.experimental.pallas import tpu as pltpu
from jax.experimental.pallas import tpu_sc as plsc

NN = 2048
BS = 4
NODES = NN * BS
HEADS = 4
GD = 16
HID = HEADS * GD
EDGES = 262144
NWORK = 32
EPW = EDGES // NWORK
CHUNK = 64
NCHUNK = EPW // CHUNK
SW = 128
DW = 128


def _prep_tables(h, w_ref, asb_ref, adb_ref, rep, s_ref, d_ref, selfm_ref):
    hW = jnp.dot(h, w_ref[...].T, preferred_element_type=jnp.float32)
    a1 = jnp.dot(hW, asb_ref[...].T, preferred_element_type=jnp.float32)
    a2 = jnp.dot(hW, adb_ref[...].T, preferred_element_type=jnp.float32)
    u1 = jnp.exp(a1)
    u2 = jnp.exp(0.2 * a1)
    v1 = jnp.exp(a2)
    v2 = jnp.exp(0.2 * a2)
    s_ref[...] = jnp.concatenate(
        [hW, u1, u2, jnp.zeros((NODES, SW - HID - 8), jnp.float32)], axis=1)
    d_ref[...] = jnp.concatenate(
        [v1, v2, jnp.zeros((NODES, DW - 8), jnp.float32)], axis=1)
    u1b = jnp.dot(u1, rep, preferred_element_type=jnp.float32)
    u2b = jnp.dot(u2, rep, preferred_element_type=jnp.float32)
    v1b = jnp.dot(v1, rep, preferred_element_type=jnp.float32)
    v2b = jnp.dot(v2, rep, preferred_element_type=jnp.float32)
    selfm_ref[...] = jnp.maximum(u1b * v1b, u2b * v2b)


_PREP_OUT = (
    jax.ShapeDtypeStruct((NODES, SW), jnp.float32),
    jax.ShapeDtypeStruct((NODES, DW), jnp.float32),
    jax.ShapeDtypeStruct((NODES, HID), jnp.float32),
)


def _prep0_body(xe_ref, fcw_ref, fcb_ref, w_ref, asb_ref, adb_ref, rep_ref,
                s_ref, d_ref, selfm_ref):
    h = jnp.dot(xe_ref[...], fcw_ref[...].T,
                preferred_element_type=jnp.float32) + fcb_ref[...]
    _prep_tables(h, w_ref, asb_ref, adb_ref, rep_ref[...],
                 s_ref, d_ref, selfm_ref)


def _prep0(xe, fc_w, fc_b, W, asb, adb, rep):
    return pl.pallas_call(
        _prep0_body, out_shape=_PREP_OUT,
    )(xe, fc_w, fc_b, W, asb, adb, rep)


def _combine_h(p_ref, s_ref, selfm_ref, rep, b_ref):
    num = (p_ref[0, :, :HID] + p_ref[1, :, :HID]
           + s_ref[:, :HID] * selfm_ref[...])
    den4 = p_ref[0, :, HID:HID + 4] + p_ref[1, :, HID:HID + 4]
    den = jnp.dot(den4, rep, preferred_element_type=jnp.float32) + selfm_ref[...]
    o = num / (den + 1e-16) + b_ref[...]
    return jnp.where(o > 0, o, 0.01 * o)


def _prepn_body(p_ref, sp_ref, selfmp_ref, b_ref, w_ref, asb_ref, adb_ref,
                rep_ref, s_ref, d_ref, selfm_ref):
    rep = rep_ref[...]
    h = _combine_h(p_ref, sp_ref, selfmp_ref, rep, b_ref)
    _prep_tables(h, w_ref, asb_ref, adb_ref, rep, s_ref, d_ref, selfm_ref)


def _prepn(p, s_prev, selfm_prev, b, W, asb, adb, rep):
    return pl.pallas_call(
        _prepn_body, out_shape=_PREP_OUT,
    )(p, s_prev, selfm_prev, b, W, asb, adb, rep)


def _dyngather(x, idx):
    return lax.gather(
        x, idx[:, None],
        lax.GatherDimensionNumbers(offset_dims=(), collapsed_slice_dims=(0,),
                                   start_index_map=(0,)),
        (1,), mode=lax.GatherScatterMode.PROMISE_IN_BOUNDS)


def _sc_edge_body(s_hbm, d_hbm, e_hbm, out_hbm,
                  etile, schunk, dchunk, sbuf0, sbuf1, dbuf0, dbuf1,
                  obuf0, obuf1, acc,
                  sem_s0, sem_s1, sem_d0, sem_d1, sem_o0, sem_o1):
    cid = lax.axis_index("c")
    sid = lax.axis_index("s")
    wid = cid * 16 + sid
    sbufs = (sbuf0, sbuf1)
    dbufs = (dbuf0, dbuf1)
    obufs = (obuf0, obuf1)
    sem_ss = (sem_s0, sem_s1)
    sem_ds = (sem_d0, sem_d1)
    sem_os = (sem_o0, sem_o1)

    iota = lax.iota(jnp.int32, 16)
    shift4 = jnp.minimum(iota + 4, 15)

    zero16 = jnp.zeros((16,), jnp.float32)

    def _zrow(e, carry):
        for j in range(8):
            obuf0[e, pl.ds(16 * j, 16)] = zero16
        return carry

    lax.fori_loop(0, CHUNK, _zrow, 0)
    for t in range(512 // CHUNK):
        pltpu.sync_copy(obuf0, acc.at[pl.ds(sid * 512 + t * CHUNK, CHUNK)])
    plsc.subcore_barrier()

    pltpu.sync_copy(e_hbm.at[wid], etile)

    def _unpack(q, row, off):
        for k in range(4):
            ev = etile[row, pl.ds(off + 16 * k, 16)]
            schunk[lax.rem(q, 2), pl.ds(16 * k, 16)] = (
                lax.bitwise_and(ev, 8191))
            dchunk[lax.rem(q, 4), pl.ds(16 * k, 16)] = (
                lax.shift_right_logical(ev, 13))

    def _issue(q, b):
        pltpu.async_copy(s_hbm.at[schunk.at[lax.rem(q, 2)]],
                         sbufs[b], sem_ss[b])
        pltpu.async_copy(d_hbm.at[dchunk.at[lax.rem(q, 4)]],
                         dbufs[b], sem_ds[b])

    _unpack(jnp.int32(0), jnp.int32(0), 0)
    _issue(jnp.int32(0), 0)

    def _outer(c2, carry):
        for b in range(2):
            c = 2 * c2 + b
            nb = 1 - b

            @pl.when(c + 1 < NCHUNK)
            def _():
                nrow = c2 + b
                _unpack(c + 1, nrow, 64 * nb)
                _issue(c + 1, nb)

            @pl.when(c >= 2)
            def _():
                pltpu.make_async_copy(
                    obufs[b], acc.at[dchunk.at[lax.rem(c, 4)]],
                    sem_os[b]).wait()

            pltpu.make_async_copy(
                s_hbm.at[schunk.at[lax.rem(c, 2)]], sbufs[b],
                sem_ss[b]).wait()
            pltpu.make_async_copy(
                d_hbm.at[dchunk.at[lax.rem(c, 4)]], dbufs[b],
                sem_ds[b]).wait()
            sbuf = sbufs[b]
            dbuf = dbufs[b]
            obuf = obufs[b]

            def _grp(g, gcarry):
                ebase = 16 * g
                for l in range(16):
                    e = ebase + l
                    dv = dbuf[e, pl.ds(0, 16)]
                    su = sbuf[e, pl.ds(HID, 16)]
                    prod = su * dv
                    m = jnp.maximum(prod, _dyngather(prod, shift4))
                    obuf[e, pl.ds(HID, 16)] = m
                    for j in range(4):
                        exj = _dyngather(m, jnp.full((16,), j, jnp.int32))
                        obuf[e, pl.ds(16 * j, 16)] = (
                            sbuf[e, pl.ds(16 * j, 16)] * exj)
                return gcarry

            lax.fori_loop(0, CHUNK // 16, _grp, 0)
            pltpu.async_copy(obuf, acc.at[dchunk.at[lax.rem(c, 4)]],
                             sem_os[b], add=True)
        return carry

    lax.fori_loop(0, NCHUNK // 2, _outer, 0)
    for b in range(2):
        c = NCHUNK - 2 + b
        pltpu.make_async_copy(
            obufs[b], acc.at[dchunk.at[lax.rem(jnp.int32(c), 4)]],
            sem_os[b]).wait()
    plsc.subcore_barrier()

    pltpu.sync_copy(acc.at[pl.ds(sid * 512, 512)],
                    out_hbm.at[cid, pl.ds(sid * 512, 512)])


def _sc_edge(s_tab, d_tab, edges_w):
    mesh = plsc.VectorSubcoreMesh(core_axis_name="c", subcore_axis_name="s")
    f = pl.kernel(
        _sc_edge_body,
        out_type=jax.ShapeDtypeStruct((2, NODES, DW), jnp.float32),
        mesh=mesh,
        scratch_types=[
            pltpu.VMEM((NCHUNK // 2, 2 * CHUNK), jnp.int32),
            pltpu.VMEM((2, CHUNK), jnp.int32),
            pltpu.VMEM((4, CHUNK), jnp.int32),
            pltpu.VMEM((CHUNK, SW), jnp.float32),
            pltpu.VMEM((CHUNK, SW), jnp.float32),
            pltpu.VMEM((CHUNK, DW), jnp.float32),
            pltpu.VMEM((CHUNK, DW), jnp.float32),
            pltpu.VMEM((CHUNK, DW), jnp.float32),
            pltpu.VMEM((CHUNK, DW), jnp.float32),
            pltpu.VMEM_SHARED((NODES, DW), jnp.float32),
            pltpu.SemaphoreType.DMA,
            pltpu.SemaphoreType.DMA,
            pltpu.SemaphoreType.DMA,
            pltpu.SemaphoreType.DMA,
            pltpu.SemaphoreType.DMA,
            pltpu.SemaphoreType.DMA,
        ],
    )
    return f(s_tab, d_tab, edges_w)


def _mha_body(p_ref, s_ref, selfm_ref, rep_ref, gb_ref,
              wi_ref, bi_ref, dow_ref, dob_ref, pw_ref, pb_ref, o_ref):
    x = _combine_h(p_ref, s_ref, selfm_ref, rep_ref[...], gb_ref)
    qkv = jnp.dot(x, wi_ref[...].T,
                  preferred_element_type=jnp.float32) + bi_ref[...]
    ohs = []
    for h in range(HEADS):
        q = qkv[:, 16 * h:16 * h + 16]
        k = qkv[:, HID + 16 * h:HID + 16 * h + 16]
        v = qkv[:, 2 * HID + 16 * h:2 * HID + 16 * h + 16]
        sc = jnp.dot(q, k.T, preferred_element_type=jnp.float32) * 0.25
        m = jnp.max(sc, axis=-1, keepdims=True)
        e = jnp.exp(sc - m)
        at = e / jnp.sum(e, axis=-1, keepdims=True)
        ohs.append(jnp.dot(at, v, preferred_element_type=jnp.float32))
    o = jnp.concatenate(ohs, axis=1)
    y = jnp.dot(o, dow_ref[...].T,
                preferred_element_type=jnp.float32) + dob_ref[...]
    o_ref[0] = jnp.dot(y, pw_ref[...].T,
                       preferred_element_type=jnp.float32) + pb_ref[...]


def _mha(p, s_tab, selfm, rep, gb, in_w, in_b, dow, dob, pw, pb):
    return pl.pallas_call(
        _mha_body,
        grid=(BS,),
        in_specs=[
            pl.BlockSpec((2, NN, DW), lambda b: (0, b, 0)),
            pl.BlockSpec((NN, SW), lambda b: (b, 0)),
            pl.BlockSpec((NN, HID), lambda b: (b, 0)),
            pl.BlockSpec((HEADS, HID), lambda b: (0, 0)),
            pl.BlockSpec((HID,), lambda b: (0,)),
            pl.BlockSpec((3 * HID, HID), lambda b: (0, 0)),
            pl.BlockSpec((3 * HID,), lambda b: (0,)),
            pl.BlockSpec((HID, HID), lambda b: (0, 0)),
            pl.BlockSpec((HID,), lambda b: (0,)),
            pl.BlockSpec((4, HID), lambda b: (0, 0)),
            pl.BlockSpec((4,), lambda b: (0,)),
        ],
        out_specs=pl.BlockSpec((1, NN, 4), lambda b: (b, 0, 0)),
        out_shape=jax.ShapeDtypeStruct((BS, NN, 4), jnp.float32),
    )(p, s_tab, selfm, rep, gb, in_w, in_b, dow, dob, pw, pb)


def _outmm_body(pz_ref, ow_ref, ob_ref, z_ref):
    z_ref[...] = jnp.dot(pz_ref[...], ow_ref[...].T,
                         preferred_element_type=jnp.float32) + ob_ref[...]


def _outmm(p, ow, ob):
    return pl.pallas_call(
        _outmm_body,
        out_shape=jax.ShapeDtypeStruct((BS, 128), jnp.float32),
    )(p.reshape(BS, NN * 4), ow, ob)


def kernel(x, edge_index, batch, emb, fc_w, fc_b,
           g0_W, g0_as, g0_ad, g0_b,
           g1_W, g1_as, g1_ad, g1_b,
           g2_W, g2_as, g2_ad, g2_b,
           g3_W, g3_as, g3_ad, g3_b,
           dec_in_w, dec_in_b, dec_out_w, dec_out_b,
           proj_w, proj_b, out_w, out_b):
    embt = jnp.broadcast_to(emb[None], (BS, NN, 4)).reshape(NODES, 4)
    xe = jnp.concatenate([x, embt], axis=1)
    edges_w = (edge_index[0] | (edge_index[1] << 13)).reshape(
        NWORK, NCHUNK // 2, 2 * CHUNK)
    eye = jnp.eye(HEADS, dtype=jnp.float32)
    rep = (eye[:, :, None] *
           jnp.ones((1, HEADS, GD), jnp.float32)).reshape(HEADS, HID)

    layers = ((g0_W, g0_as, g0_ad, g0_b), (g1_W, g1_as, g1_ad, g1_b),
              (g2_W, g2_as, g2_ad, g2_b), (g3_W, g3_as, g3_ad, g3_b))
    asbs = [(eye[:, :, None] * a_s[None, :, :]).reshape(HEADS, HID)
            for (_, a_s, _, _) in layers]
    adbs = [(eye[:, :, None] * a_d[None, :, :]).reshape(HEADS, HID)
            for (_, _, a_d, _) in layers]

    s_tab, d_tab, selfm = _prep0(xe, fc_w, fc_b, layers[0][0],
                                 asbs[0], adbs[0], rep)
    for i in range(4):
        p = _sc_edge(s_tab, d_tab, edges_w)
        b = layers[i][3]
        if i < 3:
            s_tab, d_tab, selfm = _prepn(p, s_tab, selfm, b,
                                         layers[i + 1][0], asbs[i + 1],
                                         adbs[i + 1], rep)
        else:
            pp = _mha(p, s_tab, selfm, rep, b, dec_in_w, dec_in_b,
                      dec_out_w, dec_out_b, proj_w, proj_b)

    return _outmm(pp, out_w, out_b)
